# baseline (speedup 1.0000x reference)
import jax
import jax.numpy as jnp
from jax import lax
from jax.experimental import pallas as pl
from jax.experimental.pallas import tpu as pltpu
from jax.experimental.pallas import tpu_sc as plsc

N = 10000
DH = 128
T = 4
TILES = 16
CORES = 2
GE = 128
RBLK = 1000

NSPLIT = 5120
GB = 88
CAP = TILES * GB * GE
BROWS = NSPLIT // TILES
AZR = 328
TRASH = 5184
NB = 4

G = 160
EPAD = TILES * G * GE
ZROWS = 632
NPAD = TILES * ZROWS

_MESH = plsc.VectorSubcoreMesh(core_axis_name="c", subcore_axis_name="s")


def _agg_body(tt, srcg, dstg, zrows, oo, src_v, dst_v, rows_v, acc, gsem, ssem):
    c = lax.axis_index("c")
    s = lax.axis_index("s")
    pltpu.sync_copy(zrows, acc.at[pl.ds(s * AZR, AZR)])
    plsc.subcore_barrier()
    for p in range(2):
        pltpu.sync_copy(srcg.at[c, p, s], src_v)
        pltpu.sync_copy(dstg.at[p, s], dst_v)

        def macro(m, carry):
            base = NB * m
            gds = [
                pltpu.async_copy(tt.at[src_v.at[base + k]], rows_v.at[k],
                                 gsem.at[k])
                for k in range(NB)
            ]
            sds = []
            for k in range(NB):
                gds[k].wait()
                sds.append(pltpu.async_copy(
                    rows_v.at[k], acc.at[dst_v.at[base + k]], ssem, add=True))
            for d in sds:
                d.wait()
            return carry

        lax.fori_loop(0, GB // NB, macro, 0)
        plsc.subcore_barrier()
        pltpu.sync_copy(acc.at[pl.ds(s * BROWS, BROWS)],
                        oo.at[c, pl.ds(p * NSPLIT + s * BROWS, BROWS)])
        plsc.subcore_barrier()
        if p == 0:
            pltpu.sync_copy(zrows, acc.at[pl.ds(s * AZR, AZR)])
            plsc.subcore_barrier()


_agg = pl.kernel(
    _agg_body,
    out_type=jax.ShapeDtypeStruct((CORES, 2 * NSPLIT, DH), jnp.float32),
    mesh=_MESH,
    compiler_params=pltpu.CompilerParams(use_tc_tiling_on_sc=False),
    scratch_types=[
        pltpu.VMEM((GB, GE), jnp.int32),
        pltpu.VMEM((GB, GE), jnp.int32),
        pltpu.VMEM((NB, GE, DH), jnp.float32),
        pltpu.VMEM_SHARED((TILES * AZR, DH), jnp.float32),
        pltpu.SemaphoreType.DMA((NB,)),
        pltpu.SemaphoreType.DMA,
    ],
)


def _deg_body(degidx, onerow, z16, dd, idx_v, ones_v, acc):
    c = lax.axis_index("c")
    s = lax.axis_index("s")
    pltpu.sync_copy(degidx.at[c, s], idx_v)
    pltpu.sync_copy(onerow, ones_v)
    pltpu.sync_copy(z16, acc.at[pl.ds(s * ZROWS, ZROWS)])
    plsc.subcore_barrier()

    def step(g, carry):
        pltpu.sync_copy(ones_v, acc.at[idx_v.at[g]], add=True)
        return carry

    lax.fori_loop(0, G, step, 0)
    plsc.subcore_barrier()
    pltpu.sync_copy(acc.at[pl.ds(s * ZROWS, ZROWS)],
                    dd.at[c, pl.ds(s * ZROWS, ZROWS)])


_deg = pl.kernel(
    _deg_body,
    out_type=jax.ShapeDtypeStruct((CORES, NPAD, 16), jnp.float32),
    mesh=_MESH,
    compiler_params=pltpu.CompilerParams(use_tc_tiling_on_sc=False),
    scratch_types=[
        pltpu.VMEM((G, GE), jnp.int32),
        pltpu.VMEM((GE, 16), jnp.float32),
        pltpu.VMEM_SHARED((NPAD, 16), jnp.float32),
    ],
)


def _nrm(d):
    return lax.rsqrt(jnp.maximum(d, 1.0))


def _matpair(a0, a1, di, W, b):
    nd = _nrm(di[...])
    Wv = W[...]
    return (jnp.dot(a0[0] * nd, Wv[:DH], preferred_element_type=jnp.float32)
            + jnp.dot(a1[0] * nd, Wv[DH:], preferred_element_type=jnp.float32)
            + b[...])


def _mid_body(a0, a1, di, do, W, b, o):
    g = jnp.maximum(_matpair(a0, a1, di, W, b), 0.0)
    gs = g * _nrm(do[...])
    o[0] = gs[:, :DH]
    o[1] = gs[:, DH:]


def _ru3_body(a0, a1, di, do, W, b, h, z_o, hrs_o):
    ru = jax.nn.sigmoid(_matpair(a0, a1, di, W, b))
    r = ru[:, :DH]
    z_o[...] = ru[:, DH:]
    hrs_o[...] = h[...] * r * _nrm(do[...])


def _rh3_body(a0, a1, di, do, W, b, h, z, hn_o, hns_o):
    hc = jnp.tanh(_matpair(a0, a1, di, W, b))
    zv = z[...]
    hn = zv * h[...] + (1.0 - zv) * hc
    hn_o[...] = hn
    hns_o[...] = hn * _nrm(do[...])


def _pre_body(h0, Xs, do, h0s_o, Xss_o):
    ns = _nrm(do[...])
    h0s_o[...] = h0[...] * ns
    Xss_o[...] = Xs[...] * ns[None]


_row = pl.BlockSpec((RBLK, DH), lambda i: (i, 0))
_col1 = pl.BlockSpec((RBLK, 1), lambda i: (i, 0))
_half0 = pl.BlockSpec((1, RBLK, DH), lambda i: (0, i, 0))
_half1 = pl.BlockSpec((1, RBLK, DH), lambda i: (1, i, 0))
_pair = pl.BlockSpec((2, RBLK, DH), lambda i: (0, i, 0))
_full = lambda shape: pl.BlockSpec(shape, lambda i: tuple(0 for _ in shape))
_GRID = N // RBLK

_mid = pl.pallas_call(
    _mid_body,
    grid=(_GRID,),
    in_specs=[_half0, _half1, _col1, _col1, _full((2 * DH, 2 * DH)),
              _full((1, 2 * DH))],
    out_specs=[_pair],
    out_shape=[jax.ShapeDtypeStruct((2, N, DH), jnp.float32)],
)

_ru3 = pl.pallas_call(
    _ru3_body,
    grid=(_GRID,),
    in_specs=[_half0, _half1, _col1, _col1, _full((2 * DH, 2 * DH)),
              _full((1, 2 * DH)), _row],
    out_specs=[_row, _row],
    out_shape=[jax.ShapeDtypeStruct((N, DH), jnp.float32)] * 2,
)

_rh3 = pl.pallas_call(
    _rh3_body,
    grid=(_GRID,),
    in_specs=[_half0, _half1, _col1, _col1, _full((2 * DH, DH)),
              _full((1, DH)), _row, _row],
    out_specs=[_row, _row],
    out_shape=[jax.ShapeDtypeStruct((N, DH), jnp.float32)] * 2,
)

_pre = pl.pallas_call(
    _pre_body,
    grid=(_GRID,),
    in_specs=[_row, pl.BlockSpec((T, RBLK, DH), lambda i: (0, i, 0)), _col1],
    out_specs=[_row, pl.BlockSpec((T, RBLK, DH), lambda i: (0, i, 0))],
    out_shape=[jax.ShapeDtypeStruct((N, DH), jnp.float32),
               jax.ShapeDtypeStruct((T, N, DH), jnp.float32)],
)


def _build_buckets(src, dst):
    E = src.shape[0]
    in1 = dst >= NSPLIT
    pos0 = jnp.cumsum(jnp.where(in1, 0, 1)) - 1
    pos1 = jnp.cumsum(jnp.where(in1, 1, 0)) - 1
    pos = jnp.where(in1, CAP + pos1, pos0)
    pos = jnp.where((jnp.where(in1, pos1, pos0) >= CAP), 2 * CAP, pos)
    dst_rel = jnp.where(in1, dst - NSPLIT, dst)
    srcb = jnp.zeros((2 * CAP + 1,), jnp.int32).at[pos].set(src)
    dstb = jnp.full((2 * CAP + 1,), TRASH, jnp.int32).at[pos].set(dst_rel)
    srcb = srcb[:2 * CAP].reshape(2, TILES, GB, GE)
    dstb = dstb[:2 * CAP].reshape(2, TILES, GB, GE)
    srcg = jnp.stack([srcb, srcb + N])
    return srcg, dstb


def kernel(Xs, h0, ru_W1, ru_b1, ru_W2, ru_b2, ru_W3, ru_b3,
           rh_W1, rh_b1, rh_W2, rh_b2, rh_W3, rh_b3, edge_index):
    src = edge_index[0].astype(jnp.int32)
    dst = edge_index[1].astype(jnp.int32)
    E = src.shape[0]

    srcg, dstg = _build_buckets(src, dst)
    zrows = jnp.zeros((AZR, DH), jnp.float32)

    pad = EPAD - E
    src_pN = jnp.concatenate([src, jnp.full((pad,), N, jnp.int32)])
    dst_pN = jnp.concatenate([dst, jnp.full((pad,), N, jnp.int32)])
    degidx = jnp.stack([src_pN, dst_pN]).reshape(CORES, TILES, G, GE)
    z16 = jnp.zeros((ZROWS, 16), jnp.float32)
    onerow = jnp.zeros((GE, 16), jnp.float32).at[:, 0].set(1.0)

    dd = _deg(degidx, onerow, z16)
    deg_out = dd[0, :N, 0:1]
    deg_in = dd[1, :N, 0:1]

    b_ru1 = ru_b1.reshape(1, -1)
    b_ru2 = ru_b2.reshape(1, -1)
    b_ru3 = ru_b3.reshape(1, -1)
    b_rh1 = rh_b1.reshape(1, -1)
    b_rh2 = rh_b2.reshape(1, -1)
    b_rh3 = rh_b3.reshape(1, -1)

    h_s, Xss = _pre(h0, Xs, deg_out)

    def agg(table):
        return _agg(table.reshape(2 * N, DH), srcg, dstg, zrows)

    def gnn2(t0, t1, W1, b1, W2, b2):
        a = agg(jnp.concatenate([t0, t1], axis=0))
        (gg,) = _mid(a, a, deg_in, deg_out, W1, b1)
        a = agg(gg.reshape(2 * N, DH))
        (gg,) = _mid(a, a, deg_in, deg_out, W2, b2)
        return agg(gg.reshape(2 * N, DH))

    h = h0
    hs = []
    for t in range(T):
        xst = Xss[t]
        a = gnn2(h_s, xst, ru_W1, b_ru1, ru_W2, b_ru2)
        z, hrs = _ru3(a, a, deg_in, deg_out, ru_W3, b_ru3, h)
        a = gnn2(hrs, xst, rh_W1, b_rh1, rh_W2, b_rh2)
        h, h_s = _rh3(a, a, deg_in, deg_out, rh_W3, b_rh3, h, z)
        hs.append(h)
    return jnp.stack(hs)


# speedup vs baseline: 6.0461x; 6.0461x over previous
import jax
import jax.numpy as jnp
from jax import lax
from jax.experimental import pallas as pl
from jax.experimental.pallas import tpu as pltpu
from jax.experimental.pallas import tpu_sc as plsc

N = 10000
DH = 128
T = 4
TILES = 16
CORES = 2
GE = 128
RBLK = 1000

NSPLIT = 5120
GB = 88
CAP = TILES * GB * GE
BROWS = NSPLIT // TILES
AZR = 328
TRASH = 5184
NB = 4

G = 160
EPAD = TILES * G * GE
ZROWS = 632
NPAD = TILES * ZROWS

_MESH = plsc.VectorSubcoreMesh(core_axis_name="c", subcore_axis_name="s")


def _agg_body(tt, srcg, dstg, zrows, oo, src_v, dst_v, rows_v, acc, gsem, ssem):
    c = lax.axis_index("c")
    s = lax.axis_index("s")
    pltpu.sync_copy(zrows, acc.at[pl.ds(s * AZR, AZR)])
    plsc.subcore_barrier()
    for p in range(2):
        pltpu.sync_copy(srcg.at[c, p, s], src_v)
        pltpu.sync_copy(dstg.at[p, s], dst_v)

        def macro(m, carry):
            base = NB * m
            gds = [
                pltpu.async_copy(tt.at[src_v.at[base + k]], rows_v.at[k],
                                 gsem.at[k])
                for k in range(NB)
            ]
            sds = []
            for k in range(NB):
                gds[k].wait()
                sds.append(pltpu.async_copy(
                    rows_v.at[k], acc.at[dst_v.at[base + k]], ssem, add=True))
            for d in sds:
                d.wait()
            return carry

        lax.fori_loop(0, GB // NB, macro, 0)
        plsc.subcore_barrier()
        pltpu.sync_copy(acc.at[pl.ds(s * BROWS, BROWS)],
                        oo.at[c, pl.ds(p * NSPLIT + s * BROWS, BROWS)])
        plsc.subcore_barrier()
        if p == 0:
            pltpu.sync_copy(zrows, acc.at[pl.ds(s * AZR, AZR)])
            plsc.subcore_barrier()


_agg = pl.kernel(
    _agg_body,
    out_type=jax.ShapeDtypeStruct((CORES, 2 * NSPLIT, DH), jnp.float32),
    mesh=_MESH,
    compiler_params=pltpu.CompilerParams(use_tc_tiling_on_sc=False),
    scratch_types=[
        pltpu.VMEM((GB, GE), jnp.int32),
        pltpu.VMEM((GB, GE), jnp.int32),
        pltpu.VMEM((NB, GE, DH), jnp.float32),
        pltpu.VMEM_SHARED((TILES * AZR, DH), jnp.float32),
        pltpu.SemaphoreType.DMA((NB,)),
        pltpu.SemaphoreType.DMA,
    ],
)


def _deg_body(degidx, onerow, z16, dd, idx_v, ones_v, acc):
    c = lax.axis_index("c")
    s = lax.axis_index("s")
    pltpu.sync_copy(degidx.at[c, s], idx_v)
    pltpu.sync_copy(onerow, ones_v)
    pltpu.sync_copy(z16, acc.at[pl.ds(s * ZROWS, ZROWS)])
    plsc.subcore_barrier()

    def step(g, carry):
        pltpu.sync_copy(ones_v, acc.at[idx_v.at[g]], add=True)
        return carry

    lax.fori_loop(0, G, step, 0)
    plsc.subcore_barrier()
    pltpu.sync_copy(acc.at[pl.ds(s * ZROWS, ZROWS)],
                    dd.at[c, pl.ds(s * ZROWS, ZROWS)])


_deg = pl.kernel(
    _deg_body,
    out_type=jax.ShapeDtypeStruct((CORES, NPAD, 16), jnp.float32),
    mesh=_MESH,
    compiler_params=pltpu.CompilerParams(use_tc_tiling_on_sc=False),
    scratch_types=[
        pltpu.VMEM((G, GE), jnp.int32),
        pltpu.VMEM((GE, 16), jnp.float32),
        pltpu.VMEM_SHARED((NPAD, 16), jnp.float32),
    ],
)


def _nrm(d):
    return lax.rsqrt(jnp.maximum(d, 1.0))


def _matpair(a0, a1, di, W, b):
    nd = _nrm(di[...])
    Wv = W[...]
    return (jnp.dot(a0[0] * nd, Wv[:DH], preferred_element_type=jnp.float32)
            + jnp.dot(a1[0] * nd, Wv[DH:], preferred_element_type=jnp.float32)
            + b[...])


def _mid_body(a0, a1, di, do, W, b, o):
    g = jnp.maximum(_matpair(a0, a1, di, W, b), 0.0)
    gs = g * _nrm(do[...])
    o[0] = gs[:, :DH]
    o[1] = gs[:, DH:]


def _ru3_body(a0, a1, di, do, W, b, h, z_o, hrs_o):
    ru = jax.nn.sigmoid(_matpair(a0, a1, di, W, b))
    r = ru[:, :DH]
    z_o[...] = ru[:, DH:]
    hrs_o[...] = h[...] * r * _nrm(do[...])


def _rh3_body(a0, a1, di, do, W, b, h, z, hn_o, hns_o):
    hc = jnp.tanh(_matpair(a0, a1, di, W, b))
    zv = z[...]
    hn = zv * h[...] + (1.0 - zv) * hc
    hn_o[...] = hn
    hns_o[...] = hn * _nrm(do[...])


def _pre_body(h0, Xs, do, h0s_o, Xss_o):
    ns = _nrm(do[...])
    h0s_o[...] = h0[...] * ns
    Xss_o[...] = Xs[...] * ns[None]


_row = pl.BlockSpec((RBLK, DH), lambda i: (i, 0))
_col1 = pl.BlockSpec((RBLK, 1), lambda i: (i, 0))
_half0 = pl.BlockSpec((1, RBLK, DH), lambda i: (0, i, 0))
_half1 = pl.BlockSpec((1, RBLK, DH), lambda i: (1, i, 0))
_pair = pl.BlockSpec((2, RBLK, DH), lambda i: (0, i, 0))
_full = lambda shape: pl.BlockSpec(shape, lambda i: tuple(0 for _ in shape))
_GRID = N // RBLK

_mid = pl.pallas_call(
    _mid_body,
    grid=(_GRID,),
    in_specs=[_half0, _half1, _col1, _col1, _full((2 * DH, 2 * DH)),
              _full((1, 2 * DH))],
    out_specs=[_pair],
    out_shape=[jax.ShapeDtypeStruct((2, N, DH), jnp.float32)],
)

_ru3 = pl.pallas_call(
    _ru3_body,
    grid=(_GRID,),
    in_specs=[_half0, _half1, _col1, _col1, _full((2 * DH, 2 * DH)),
              _full((1, 2 * DH)), _row],
    out_specs=[_row, _row],
    out_shape=[jax.ShapeDtypeStruct((N, DH), jnp.float32)] * 2,
)

_rh3 = pl.pallas_call(
    _rh3_body,
    grid=(_GRID,),
    in_specs=[_half0, _half1, _col1, _col1, _full((2 * DH, DH)),
              _full((1, DH)), _row, _row],
    out_specs=[_row, _row],
    out_shape=[jax.ShapeDtypeStruct((N, DH), jnp.float32)] * 2,
)

_pre = pl.pallas_call(
    _pre_body,
    grid=(_GRID,),
    in_specs=[_row, pl.BlockSpec((T, RBLK, DH), lambda i: (0, i, 0)), _col1],
    out_specs=[_row, pl.BlockSpec((T, RBLK, DH), lambda i: (0, i, 0))],
    out_shape=[jax.ShapeDtypeStruct((N, DH), jnp.float32),
               jax.ShapeDtypeStruct((T, N, DH), jnp.float32)],
)


def _build_buckets(src, dst):
    E = src.shape[0]
    in1 = dst >= NSPLIT
    pos0 = jnp.cumsum(jnp.where(in1, 0, 1)) - 1
    pos1 = jnp.cumsum(jnp.where(in1, 1, 0)) - 1
    pos = jnp.where(in1, CAP + pos1, pos0)
    pos = jnp.where((jnp.where(in1, pos1, pos0) >= CAP), 2 * CAP, pos)
    dst_rel = jnp.where(in1, dst - NSPLIT, dst)
    j = jnp.arange(2 * CAP + 1, dtype=jnp.int32)
    srcb = (j % N).at[pos].set(src)
    dstb = (NSPLIT + (j % 128)).at[pos].set(dst_rel)
    srcb = srcb[:2 * CAP].reshape(2, TILES, GB, GE)
    dstb = dstb[:2 * CAP].reshape(2, TILES, GB, GE)
    srcg = jnp.stack([srcb, srcb + N])
    return srcg, dstb


def kernel(Xs, h0, ru_W1, ru_b1, ru_W2, ru_b2, ru_W3, ru_b3,
           rh_W1, rh_b1, rh_W2, rh_b2, rh_W3, rh_b3, edge_index):
    src = edge_index[0].astype(jnp.int32)
    dst = edge_index[1].astype(jnp.int32)
    E = src.shape[0]

    srcg, dstg = _build_buckets(src, dst)
    zrows = jnp.zeros((AZR, DH), jnp.float32)

    pad = EPAD - E
    padrows = N + (jnp.arange(pad, dtype=jnp.int32) % (NPAD - N))
    src_pN = jnp.concatenate([src, padrows])
    dst_pN = jnp.concatenate([dst, padrows])
    degidx = jnp.stack([src_pN, dst_pN]).reshape(CORES, TILES, G, GE)
    z16 = jnp.zeros((ZROWS, 16), jnp.float32)
    onerow = jnp.zeros((GE, 16), jnp.float32).at[:, 0].set(1.0)

    dd = _deg(degidx, onerow, z16)
    deg_out = dd[0, :N, 0:1]
    deg_in = dd[1, :N, 0:1]

    b_ru1 = ru_b1.reshape(1, -1)
    b_ru2 = ru_b2.reshape(1, -1)
    b_ru3 = ru_b3.reshape(1, -1)
    b_rh1 = rh_b1.reshape(1, -1)
    b_rh2 = rh_b2.reshape(1, -1)
    b_rh3 = rh_b3.reshape(1, -1)

    h_s, Xss = _pre(h0, Xs, deg_out)

    def agg(table):
        return _agg(table.reshape(2 * N, DH), srcg, dstg, zrows)

    def gnn2(t0, t1, W1, b1, W2, b2):
        a = agg(jnp.concatenate([t0, t1], axis=0))
        (gg,) = _mid(a, a, deg_in, deg_out, W1, b1)
        a = agg(gg.reshape(2 * N, DH))
        (gg,) = _mid(a, a, deg_in, deg_out, W2, b2)
        return agg(gg.reshape(2 * N, DH))

    h = h0
    hs = []
    for t in range(T):
        xst = Xss[t]
        a = gnn2(h_s, xst, ru_W1, b_ru1, ru_W2, b_ru2)
        z, hrs = _ru3(a, a, deg_in, deg_out, ru_W3, b_ru3, h)
        a = gnn2(hrs, xst, rh_W1, b_rh1, rh_W2, b_rh2)
        h, h_s = _rh3(a, a, deg_in, deg_out, rh_W3, b_rh3, h, z)
        hs.append(h)
    return jnp.stack(hs)


# speedup vs baseline: 6.7840x; 1.1220x over previous
import jax
import jax.numpy as jnp
from jax import lax
from jax.experimental import pallas as pl
from jax.experimental.pallas import tpu as pltpu
from jax.experimental.pallas import tpu_sc as plsc

N = 10000
DH = 128
T = 4
TILES = 16
CORES = 2
GE = 128
RBLK = 1000

NSPLIT = 5120
GB = 88
CAP = TILES * GB * GE
BROWS = NSPLIT // TILES
AZR = 328
TRASH = 5184
NB = 4

G = 160
EPAD = TILES * G * GE
ZROWS = 632
NPAD = TILES * ZROWS

_MESH = plsc.VectorSubcoreMesh(core_axis_name="c", subcore_axis_name="s")


def _agg_body(tt, srcg, dstg, zrows, oo, src_v, dst_v, rows_v, acc, gsem, ssem):
    c = lax.axis_index("c")
    s = lax.axis_index("s")
    pltpu.sync_copy(zrows, acc.at[pl.ds(s * AZR, AZR)])
    plsc.subcore_barrier()
    for p in range(2):
        pltpu.sync_copy(srcg.at[c, p, s], src_v)
        pltpu.sync_copy(dstg.at[p, s], dst_v)

        def macro(m, carry):
            base = NB * m
            gds = [
                pltpu.async_copy(tt.at[src_v.at[base + k]], rows_v.at[k],
                                 gsem.at[k])
                for k in range(NB)
            ]
            sds = []
            for k in range(NB):
                gds[k].wait()
                sds.append(pltpu.async_copy(
                    rows_v.at[k], acc.at[dst_v.at[base + k]], ssem, add=True))
            for d in sds:
                d.wait()
            return carry

        lax.fori_loop(0, GB // NB, macro, 0)
        plsc.subcore_barrier()
        pltpu.sync_copy(acc.at[pl.ds(s * BROWS, BROWS)],
                        oo.at[c, pl.ds(p * NSPLIT + s * BROWS, BROWS)])
        plsc.subcore_barrier()
        if p == 0:
            pltpu.sync_copy(zrows, acc.at[pl.ds(s * AZR, AZR)])
            plsc.subcore_barrier()


_agg = pl.kernel(
    _agg_body,
    out_type=jax.ShapeDtypeStruct((CORES, 2 * NSPLIT, DH), jnp.float32),
    mesh=_MESH,
    compiler_params=pltpu.CompilerParams(use_tc_tiling_on_sc=False),
    scratch_types=[
        pltpu.VMEM((GB, GE), jnp.int32),
        pltpu.VMEM((GB, GE), jnp.int32),
        pltpu.VMEM((NB, GE, DH), jnp.float32),
        pltpu.VMEM_SHARED((TILES * AZR, DH), jnp.float32),
        pltpu.SemaphoreType.DMA((NB,)),
        pltpu.SemaphoreType.DMA,
    ],
)


def _deg_body(degidx, onerow, z16, dd, idx_v, ones_v, acc):
    c = lax.axis_index("c")
    s = lax.axis_index("s")
    pltpu.sync_copy(degidx.at[c, s], idx_v)
    pltpu.sync_copy(onerow, ones_v)
    pltpu.sync_copy(z16, acc.at[pl.ds(s * ZROWS, ZROWS)])
    plsc.subcore_barrier()

    def step(g, carry):
        pltpu.sync_copy(ones_v, acc.at[idx_v.at[g]], add=True)
        return carry

    lax.fori_loop(0, G, step, 0)
    plsc.subcore_barrier()
    pltpu.sync_copy(acc.at[pl.ds(s * ZROWS, ZROWS)],
                    dd.at[c, pl.ds(s * ZROWS, ZROWS)])


_deg = pl.kernel(
    _deg_body,
    out_type=jax.ShapeDtypeStruct((CORES, NPAD, 16), jnp.float32),
    mesh=_MESH,
    compiler_params=pltpu.CompilerParams(use_tc_tiling_on_sc=False),
    scratch_types=[
        pltpu.VMEM((G, GE), jnp.int32),
        pltpu.VMEM((GE, 16), jnp.float32),
        pltpu.VMEM_SHARED((NPAD, 16), jnp.float32),
    ],
)


def _nrm(d):
    return lax.rsqrt(jnp.maximum(d, 1.0))


def _matpair(a0, a1, di, W, b):
    nd = _nrm(di[...])
    Wv = W[...]
    return (jnp.dot(a0[0] * nd, Wv[:DH], preferred_element_type=jnp.float32)
            + jnp.dot(a1[0] * nd, Wv[DH:], preferred_element_type=jnp.float32)
            + b[...])


def _mid_body(a0, a1, di, do, W, b, o):
    g = jnp.maximum(_matpair(a0, a1, di, W, b), 0.0)
    gs = g * _nrm(do[...])
    o[0] = gs[:, :DH]
    o[1] = gs[:, DH:]


def _ru3_body(a0, a1, di, do, W, b, h, z_o, hrs_o):
    ru = jax.nn.sigmoid(_matpair(a0, a1, di, W, b))
    r = ru[:, :DH]
    z_o[...] = ru[:, DH:]
    hrs_o[...] = h[...] * r * _nrm(do[...])


def _rh3_body(a0, a1, di, do, W, b, h, z, hn_o, hns_o):
    hc = jnp.tanh(_matpair(a0, a1, di, W, b))
    zv = z[...]
    hn = zv * h[...] + (1.0 - zv) * hc
    hn_o[...] = hn
    hns_o[...] = hn * _nrm(do[...])


def _pre_body(h0, Xs, do, h0s_o, Xss_o):
    ns = _nrm(do[...])
    h0s_o[...] = h0[...] * ns
    Xss_o[...] = Xs[...] * ns[None]


_row = pl.BlockSpec((RBLK, DH), lambda i: (i, 0))
_col1 = pl.BlockSpec((RBLK, 1), lambda i: (i, 0))
_half0 = pl.BlockSpec((1, RBLK, DH), lambda i: (0, i, 0))
_half1 = pl.BlockSpec((1, RBLK, DH), lambda i: (1, i, 0))
_pair = pl.BlockSpec((2, RBLK, DH), lambda i: (0, i, 0))
_full = lambda shape: pl.BlockSpec(shape, lambda i: tuple(0 for _ in shape))
_GRID = N // RBLK

_mid = pl.pallas_call(
    _mid_body,
    grid=(_GRID,),
    in_specs=[_half0, _half1, _col1, _col1, _full((2 * DH, 2 * DH)),
              _full((1, 2 * DH))],
    out_specs=[_pair],
    out_shape=[jax.ShapeDtypeStruct((2, N, DH), jnp.float32)],
)

_ru3 = pl.pallas_call(
    _ru3_body,
    grid=(_GRID,),
    in_specs=[_half0, _half1, _col1, _col1, _full((2 * DH, 2 * DH)),
              _full((1, 2 * DH)), _row],
    out_specs=[_row, _row],
    out_shape=[jax.ShapeDtypeStruct((N, DH), jnp.float32)] * 2,
)

_rh3 = pl.pallas_call(
    _rh3_body,
    grid=(_GRID,),
    in_specs=[_half0, _half1, _col1, _col1, _full((2 * DH, DH)),
              _full((1, DH)), _row, _row],
    out_specs=[_row, _row],
    out_shape=[jax.ShapeDtypeStruct((N, DH), jnp.float32)] * 2,
)

_pre = pl.pallas_call(
    _pre_body,
    grid=(_GRID,),
    in_specs=[_row, pl.BlockSpec((T, RBLK, DH), lambda i: (0, i, 0)), _col1],
    out_specs=[_row, pl.BlockSpec((T, RBLK, DH), lambda i: (0, i, 0))],
    out_shape=[jax.ShapeDtypeStruct((N, DH), jnp.float32),
               jax.ShapeDtypeStruct((T, N, DH), jnp.float32)],
)


def _build_buckets(src, dst):
    E = src.shape[0]
    in1 = dst >= NSPLIT
    pos0 = jnp.cumsum(jnp.where(in1, 0, 1)) - 1
    pos1 = jnp.cumsum(jnp.where(in1, 1, 0)) - 1
    pos = jnp.where(in1, CAP + pos1, pos0)
    pos = jnp.where((jnp.where(in1, pos1, pos0) >= CAP), 2 * CAP, pos)
    dst_rel = jnp.where(in1, dst - NSPLIT, dst)
    j = jnp.arange(2 * CAP + 1, dtype=jnp.int32)
    padval = (j % N) * 8192 + NSPLIT + (j % 128)
    packed = padval.at[pos].set(src * 8192 + dst_rel)
    srcb = packed >> 13
    dstb = packed & 8191
    srcb = srcb[:2 * CAP].reshape(2, TILES, GB, GE)
    dstb = dstb[:2 * CAP].reshape(2, TILES, GB, GE)
    srcg = jnp.stack([srcb, srcb + N])
    return srcg, dstb


def kernel(Xs, h0, ru_W1, ru_b1, ru_W2, ru_b2, ru_W3, ru_b3,
           rh_W1, rh_b1, rh_W2, rh_b2, rh_W3, rh_b3, edge_index):
    src = edge_index[0].astype(jnp.int32)
    dst = edge_index[1].astype(jnp.int32)
    E = src.shape[0]

    srcg, dstg = _build_buckets(src, dst)
    zrows = jnp.zeros((AZR, DH), jnp.float32)

    pad = EPAD - E
    padrows = N + (jnp.arange(pad, dtype=jnp.int32) % (NPAD - N))
    src_pN = jnp.concatenate([src, padrows])
    dst_pN = jnp.concatenate([dst, padrows])
    degidx = jnp.stack([src_pN, dst_pN]).reshape(CORES, TILES, G, GE)
    z16 = jnp.zeros((ZROWS, 16), jnp.float32)
    onerow = jnp.zeros((GE, 16), jnp.float32).at[:, 0].set(1.0)

    dd = _deg(degidx, onerow, z16)
    deg_out = dd[0, :N, 0:1]
    deg_in = dd[1, :N, 0:1]

    b_ru1 = ru_b1.reshape(1, -1)
    b_ru2 = ru_b2.reshape(1, -1)
    b_ru3 = ru_b3.reshape(1, -1)
    b_rh1 = rh_b1.reshape(1, -1)
    b_rh2 = rh_b2.reshape(1, -1)
    b_rh3 = rh_b3.reshape(1, -1)

    h_s, Xss = _pre(h0, Xs, deg_out)

    def agg(table):
        return _agg(table.reshape(2 * N, DH), srcg, dstg, zrows)

    def gnn2(t0, t1, W1, b1, W2, b2):
        a = agg(jnp.concatenate([t0, t1], axis=0))
        (gg,) = _mid(a, a, deg_in, deg_out, W1, b1)
        a = agg(gg.reshape(2 * N, DH))
        (gg,) = _mid(a, a, deg_in, deg_out, W2, b2)
        return agg(gg.reshape(2 * N, DH))

    h = h0
    hs = []
    for t in range(T):
        xst = Xss[t]
        a = gnn2(h_s, xst, ru_W1, b_ru1, ru_W2, b_ru2)
        z, hrs = _ru3(a, a, deg_in, deg_out, ru_W3, b_ru3, h)
        a = gnn2(hrs, xst, rh_W1, b_rh1, rh_W2, b_rh2)
        h, h_s = _rh3(a, a, deg_in, deg_out, rh_W3, b_rh3, h, z)
        hs.append(h)
    return jnp.stack(hs)


# speedup vs baseline: 6.7883x; 1.0006x over previous
import jax
import jax.numpy as jnp
from jax import lax
from jax.experimental import pallas as pl
from jax.experimental.pallas import tpu as pltpu
from jax.experimental.pallas import tpu_sc as plsc

N = 10000
DH = 128
T = 4
TILES = 16
CORES = 2
GE = 128
RBLK = 1000

NSPLIT = 5120
GB = 88
CAP = TILES * GB * GE
BROWS = NSPLIT // TILES
AZR = 328
TRASH = 5184
NB = 4

G = 160
EPAD = TILES * G * GE
ZROWS = 632
NPAD = TILES * ZROWS

_MESH = plsc.VectorSubcoreMesh(core_axis_name="c", subcore_axis_name="s")


def _agg_body(tt, srcg, dstg, zrows, oo, src_v, dst_v, rows_v, acc, gsem, ssem):
    c = lax.axis_index("c")
    s = lax.axis_index("s")
    pltpu.sync_copy(zrows, acc.at[pl.ds(s * AZR, AZR)])
    plsc.subcore_barrier()
    for p in range(2):
        pltpu.sync_copy(srcg.at[c, p, s], src_v)
        pltpu.sync_copy(dstg.at[p, s], dst_v)

        def macro(m, carry):
            base = NB * m
            gds = [
                pltpu.async_copy(tt.at[src_v.at[base + k]], rows_v.at[k],
                                 gsem.at[k])
                for k in range(NB)
            ]
            sds = []
            for k in range(NB):
                gds[k].wait()
                sds.append(pltpu.async_copy(
                    rows_v.at[k], acc.at[dst_v.at[base + k]], ssem, add=True))
            for d in sds:
                d.wait()
            return carry

        lax.fori_loop(0, GB // NB, macro, 0)
        plsc.subcore_barrier()
        pltpu.sync_copy(acc.at[pl.ds(s * BROWS, BROWS)],
                        oo.at[c, pl.ds(p * NSPLIT + s * BROWS, BROWS)])
        plsc.subcore_barrier()
        if p == 0:
            pltpu.sync_copy(zrows, acc.at[pl.ds(s * AZR, AZR)])
            plsc.subcore_barrier()


_agg = pl.kernel(
    _agg_body,
    out_type=jax.ShapeDtypeStruct((CORES, 2 * NSPLIT, DH), jnp.float32),
    mesh=_MESH,
    compiler_params=pltpu.CompilerParams(use_tc_tiling_on_sc=False),
    scratch_types=[
        pltpu.VMEM((GB, GE), jnp.int32),
        pltpu.VMEM((GB, GE), jnp.int32),
        pltpu.VMEM((NB, GE, DH), jnp.float32),
        pltpu.VMEM_SHARED((TILES * AZR, DH), jnp.float32),
        pltpu.SemaphoreType.DMA((NB,)),
        pltpu.SemaphoreType.DMA,
    ],
)


def _deg_body(degidx, onerow, z16, dd, idx_v, ones_v, acc, dsem):
    c = lax.axis_index("c")
    s = lax.axis_index("s")
    pltpu.sync_copy(degidx.at[c, s], idx_v)
    pltpu.sync_copy(onerow, ones_v)
    pltpu.sync_copy(z16, acc.at[pl.ds(s * ZROWS, ZROWS)])
    plsc.subcore_barrier()

    def step(m, carry):
        base = 8 * m
        sds = [
            pltpu.async_copy(ones_v, acc.at[idx_v.at[base + k]], dsem,
                             add=True)
            for k in range(8)
        ]
        for d in sds:
            d.wait()
        return carry

    lax.fori_loop(0, G // 8, step, 0)
    plsc.subcore_barrier()
    pltpu.sync_copy(acc.at[pl.ds(s * ZROWS, ZROWS)],
                    dd.at[c, pl.ds(s * ZROWS, ZROWS)])


_deg = pl.kernel(
    _deg_body,
    out_type=jax.ShapeDtypeStruct((CORES, NPAD, 16), jnp.float32),
    mesh=_MESH,
    compiler_params=pltpu.CompilerParams(use_tc_tiling_on_sc=False),
    scratch_types=[
        pltpu.VMEM((G, GE), jnp.int32),
        pltpu.VMEM((GE, 16), jnp.float32),
        pltpu.VMEM_SHARED((NPAD, 16), jnp.float32),
        pltpu.SemaphoreType.DMA,
    ],
)


def _nrm(d):
    return lax.rsqrt(jnp.maximum(d, 1.0))


def _matpair(a0, a1, di, W, b):
    nd = _nrm(di[...])
    Wv = W[...]
    return (jnp.dot(a0[0] * nd, Wv[:DH], preferred_element_type=jnp.float32)
            + jnp.dot(a1[0] * nd, Wv[DH:], preferred_element_type=jnp.float32)
            + b[...])


def _mid_body(a0, a1, di, do, W, b, o):
    g = jnp.maximum(_matpair(a0, a1, di, W, b), 0.0)
    gs = g * _nrm(do[...])
    o[0] = gs[:, :DH]
    o[1] = gs[:, DH:]


def _ru3_body(a0, a1, di, do, W, b, h, z_o, hrs_o):
    ru = jax.nn.sigmoid(_matpair(a0, a1, di, W, b))
    r = ru[:, :DH]
    z_o[...] = ru[:, DH:]
    hrs_o[...] = h[...] * r * _nrm(do[...])


def _rh3_body(a0, a1, di, do, W, b, h, z, hn_o, hns_o):
    hc = jnp.tanh(_matpair(a0, a1, di, W, b))
    zv = z[...]
    hn = zv * h[...] + (1.0 - zv) * hc
    hn_o[...] = hn
    hns_o[...] = hn * _nrm(do[...])


def _pre_body(h0, Xs, do, h0s_o, Xss_o):
    ns = _nrm(do[...])
    h0s_o[...] = h0[...] * ns
    Xss_o[...] = Xs[...] * ns[None]


_row = pl.BlockSpec((RBLK, DH), lambda i: (i, 0))
_col1 = pl.BlockSpec((RBLK, 1), lambda i: (i, 0))
_half0 = pl.BlockSpec((1, RBLK, DH), lambda i: (0, i, 0))
_half1 = pl.BlockSpec((1, RBLK, DH), lambda i: (1, i, 0))
_pair = pl.BlockSpec((2, RBLK, DH), lambda i: (0, i, 0))
_full = lambda shape: pl.BlockSpec(shape, lambda i: tuple(0 for _ in shape))
_GRID = N // RBLK

_mid = pl.pallas_call(
    _mid_body,
    grid=(_GRID,),
    in_specs=[_half0, _half1, _col1, _col1, _full((2 * DH, 2 * DH)),
              _full((1, 2 * DH))],
    out_specs=[_pair],
    out_shape=[jax.ShapeDtypeStruct((2, N, DH), jnp.float32)],
)

_ru3 = pl.pallas_call(
    _ru3_body,
    grid=(_GRID,),
    in_specs=[_half0, _half1, _col1, _col1, _full((2 * DH, 2 * DH)),
              _full((1, 2 * DH)), _row],
    out_specs=[_row, _row],
    out_shape=[jax.ShapeDtypeStruct((N, DH), jnp.float32)] * 2,
)

_rh3 = pl.pallas_call(
    _rh3_body,
    grid=(_GRID,),
    in_specs=[_half0, _half1, _col1, _col1, _full((2 * DH, DH)),
              _full((1, DH)), _row, _row],
    out_specs=[_row, _row],
    out_shape=[jax.ShapeDtypeStruct((N, DH), jnp.float32)] * 2,
)

_pre = pl.pallas_call(
    _pre_body,
    grid=(_GRID,),
    in_specs=[_row, pl.BlockSpec((T, RBLK, DH), lambda i: (0, i, 0)), _col1],
    out_specs=[_row, pl.BlockSpec((T, RBLK, DH), lambda i: (0, i, 0))],
    out_shape=[jax.ShapeDtypeStruct((N, DH), jnp.float32),
               jax.ShapeDtypeStruct((T, N, DH), jnp.float32)],
)


def _build_buckets(src, dst):
    E = src.shape[0]
    in1 = dst >= NSPLIT
    pos0 = jnp.cumsum(jnp.where(in1, 0, 1)) - 1
    pos1 = jnp.cumsum(jnp.where(in1, 1, 0)) - 1
    pos = jnp.where(in1, CAP + pos1, pos0)
    pos = jnp.where((jnp.where(in1, pos1, pos0) >= CAP), 2 * CAP, pos)
    dst_rel = jnp.where(in1, dst - NSPLIT, dst)
    j = jnp.arange(2 * CAP + 1, dtype=jnp.int32)
    padval = (j % N) * 8192 + NSPLIT + (j % 128)
    packed = padval.at[pos].set(src * 8192 + dst_rel)
    srcb = packed >> 13
    dstb = packed & 8191
    srcb = srcb[:2 * CAP].reshape(2, TILES, GB, GE)
    dstb = dstb[:2 * CAP].reshape(2, TILES, GB, GE)
    srcg = jnp.stack([srcb, srcb + N])
    return srcg, dstb


def kernel(Xs, h0, ru_W1, ru_b1, ru_W2, ru_b2, ru_W3, ru_b3,
           rh_W1, rh_b1, rh_W2, rh_b2, rh_W3, rh_b3, edge_index):
    src = edge_index[0].astype(jnp.int32)
    dst = edge_index[1].astype(jnp.int32)
    E = src.shape[0]

    srcg, dstg = _build_buckets(src, dst)
    zrows = jnp.zeros((AZR, DH), jnp.float32)

    pad = EPAD - E
    padrows = N + (jnp.arange(pad, dtype=jnp.int32) % (NPAD - N))
    src_pN = jnp.concatenate([src, padrows])
    dst_pN = jnp.concatenate([dst, padrows])
    degidx = jnp.stack([src_pN, dst_pN]).reshape(CORES, TILES, G, GE)
    z16 = jnp.zeros((ZROWS, 16), jnp.float32)
    onerow = jnp.zeros((GE, 16), jnp.float32).at[:, 0].set(1.0)

    dd = _deg(degidx, onerow, z16)
    deg_out = dd[0, :N, 0:1]
    deg_in = dd[1, :N, 0:1]

    b_ru1 = ru_b1.reshape(1, -1)
    b_ru2 = ru_b2.reshape(1, -1)
    b_ru3 = ru_b3.reshape(1, -1)
    b_rh1 = rh_b1.reshape(1, -1)
    b_rh2 = rh_b2.reshape(1, -1)
    b_rh3 = rh_b3.reshape(1, -1)

    h_s, Xss = _pre(h0, Xs, deg_out)

    def agg(table):
        return _agg(table.reshape(2 * N, DH), srcg, dstg, zrows)

    def gnn2(t0, t1, W1, b1, W2, b2):
        a = agg(jnp.concatenate([t0, t1], axis=0))
        (gg,) = _mid(a, a, deg_in, deg_out, W1, b1)
        a = agg(gg.reshape(2 * N, DH))
        (gg,) = _mid(a, a, deg_in, deg_out, W2, b2)
        return agg(gg.reshape(2 * N, DH))

    h = h0
    hs = []
    for t in range(T):
        xst = Xss[t]
        a = gnn2(h_s, xst, ru_W1, b_ru1, ru_W2, b_ru2)
        z, hrs = _ru3(a, a, deg_in, deg_out, ru_W3, b_ru3, h)
        a = gnn2(hrs, xst, rh_W1, b_rh1, rh_W2, b_rh2)
        h, h_s = _rh3(a, a, deg_in, deg_out, rh_W3, b_rh3, h, z)
        hs.append(h)
    return jnp.stack(hs)
